# baseline (device time: 122781 ns/iter reference)
import jax
import jax.numpy as jnp
from jax import lax
from jax.experimental import pallas as pl
from jax.experimental.pallas import tpu as pltpu

K = 16


def kernel(partial, gamma):
    _, m_tot, d = partial.shape
    m_half = m_tot // 2
    m_q = m_half // 2
    r = m_q // K

    def body(partial_ref, gamma_ref, out_ref, comm_ref, stage_ref,
             ysend, yrecv, xsend, xrecv, local_sem, stage_sems):
        my_x = lax.axis_index("x")
        my_y = lax.axis_index("y")
        y_nbr = (my_x, 1 - my_y)
        x_nbr = (1 - my_x, my_y)

        barrier_sem = pltpu.get_barrier_semaphore()
        for nbr in (y_nbr, x_nbr):
            pl.semaphore_signal(
                barrier_sem, inc=1, device_id=nbr,
                device_id_type=pl.DeviceIdType.MESH,
            )
        pl.semaphore_wait(barrier_sem, 2)

        send_base = (1 - my_y) * m_half + my_x * m_q
        my_base = my_y * m_half + my_x * m_q
        out_q = my_x * m_q

        def y_desc(k):
            return pltpu.make_async_remote_copy(
                src_ref=stage_ref.at[pl.ds(k * r, r), :],
                dst_ref=comm_ref.at[pl.ds(k * r, r), :],
                send_sem=ysend.at[k],
                recv_sem=yrecv.at[k],
                device_id=y_nbr,
                device_id_type=pl.DeviceIdType.MESH,
            )

        def x_out_desc(k):
            sl = pl.ds(out_q + k * r, r)
            return pltpu.make_async_remote_copy(
                src_ref=out_ref.at[sl, :],
                dst_ref=out_ref.at[sl, :],
                send_sem=xsend.at[k],
                recv_sem=xrecv.at[k],
                device_id=x_nbr,
                device_id_type=pl.DeviceIdType.MESH,
            )

        def x_in_desc(k):
            sl = pl.ds((1 - my_x) * m_q + k * r, r)
            return pltpu.make_async_remote_copy(
                src_ref=out_ref.at[sl, :],
                dst_ref=out_ref.at[sl, :],
                send_sem=xsend.at[k],
                recv_sem=xrecv.at[k],
                device_id=x_nbr,
                device_id_type=pl.DeviceIdType.MESH,
            )

        def stage_desc(k):
            return pltpu.make_async_copy(
                partial_ref.at[0, pl.ds(send_base + k * r, r), :],
                stage_ref.at[pl.ds(k * r, r), :],
                stage_sems.at[k],
            )

        for k in range(K):
            stage_desc(k).start()

        lcopy = pltpu.make_async_copy(
            partial_ref.at[0, pl.ds(my_base, m_q), :],
            out_ref.at[pl.ds(out_q, m_q), :],
            local_sem,
        )
        lcopy.start()

        for k in range(K):
            stage_desc(k).wait()
            y_desc(k).start()

        lcopy.wait()

        for k in range(K):
            y_desc(k).wait_recv()
            sl = pl.ds(out_q + k * r, r)
            y = out_ref[sl, :] + comm_ref[pl.ds(k * r, r), :]
            ms = jnp.mean(y * y, axis=-1, keepdims=True)
            out_ref[sl, :] = y * lax.rsqrt(ms + 1e-6) * gamma_ref[:]
            x_out_desc(k).start()

        for k in range(K):
            x_in_desc(k).wait_recv()
        for k in range(K):
            y_desc(k).wait_send()
            x_out_desc(k).wait_send()

    return pl.pallas_call(
        body,
        out_shape=jax.ShapeDtypeStruct((m_half, d), jnp.float32),
        in_specs=[
            pl.BlockSpec(memory_space=pl.ANY),
            pl.BlockSpec(memory_space=pltpu.VMEM),
        ],
        out_specs=pl.BlockSpec(memory_space=pltpu.VMEM),
        scratch_shapes=[
            pltpu.VMEM((m_q, d), jnp.float32),
            pltpu.VMEM((m_q, d), jnp.float32),
            pltpu.SemaphoreType.DMA((K,)),
            pltpu.SemaphoreType.DMA((K,)),
            pltpu.SemaphoreType.DMA((K,)),
            pltpu.SemaphoreType.DMA((K,)),
            pltpu.SemaphoreType.DMA,
            pltpu.SemaphoreType.DMA((K,)),
        ],
        compiler_params=pltpu.CompilerParams(
            collective_id=0,
            vmem_limit_bytes=56 * 1024 * 1024,
        ),
    )(partial, gamma)


# device time: 117351 ns/iter; 1.0463x vs baseline; 1.0463x over previous
import jax
import jax.numpy as jnp
from jax import lax
from jax.experimental import pallas as pl
from jax.experimental.pallas import tpu as pltpu

K = 16


def kernel(partial, gamma):
    _, m_tot, d = partial.shape
    m_half = m_tot // 2
    m_q = m_half // 2
    r = m_q // K

    def body(partial_ref, gamma_ref, out_ref, comm_ref, stage_ref, res_ref,
             ysend, yrecv, xsend, xrecv, local_sem, stage_sems, out_sems):
        my_x = lax.axis_index("x")
        my_y = lax.axis_index("y")
        y_nbr = (my_x, 1 - my_y)
        x_nbr = (1 - my_x, my_y)

        barrier_sem = pltpu.get_barrier_semaphore()
        for nbr in (y_nbr, x_nbr):
            pl.semaphore_signal(
                barrier_sem, inc=1, device_id=nbr,
                device_id_type=pl.DeviceIdType.MESH,
            )
        pl.semaphore_wait(barrier_sem, 2)

        send_base = (1 - my_y) * m_half + my_x * m_q
        my_base = my_y * m_half + my_x * m_q
        out_q = my_x * m_q

        def y_desc(k):
            return pltpu.make_async_remote_copy(
                src_ref=stage_ref.at[pl.ds(k * r, r), :],
                dst_ref=comm_ref.at[pl.ds(k * r, r), :],
                send_sem=ysend.at[k],
                recv_sem=yrecv.at[k],
                device_id=y_nbr,
                device_id_type=pl.DeviceIdType.MESH,
            )

        def x_out_desc(k):
            return pltpu.make_async_remote_copy(
                src_ref=res_ref.at[pl.ds(k * r, r), :],
                dst_ref=out_ref.at[pl.ds(out_q + k * r, r), :],
                send_sem=xsend.at[k],
                recv_sem=xrecv.at[k],
                device_id=x_nbr,
                device_id_type=pl.DeviceIdType.MESH,
            )

        def x_in_desc(k):
            return pltpu.make_async_remote_copy(
                src_ref=res_ref.at[pl.ds(k * r, r), :],
                dst_ref=out_ref.at[pl.ds((1 - my_x) * m_q + k * r, r), :],
                send_sem=xsend.at[k],
                recv_sem=xrecv.at[k],
                device_id=x_nbr,
                device_id_type=pl.DeviceIdType.MESH,
            )

        def out_desc(k):
            return pltpu.make_async_copy(
                res_ref.at[pl.ds(k * r, r), :],
                out_ref.at[pl.ds(out_q + k * r, r), :],
                out_sems.at[k],
            )

        def stage_desc(k):
            return pltpu.make_async_copy(
                partial_ref.at[0, pl.ds(send_base + k * r, r), :],
                stage_ref.at[pl.ds(k * r, r), :],
                stage_sems.at[k],
            )

        for k in range(K):
            stage_desc(k).start()

        lcopy = pltpu.make_async_copy(
            partial_ref.at[0, pl.ds(my_base, m_q), :],
            res_ref,
            local_sem,
        )
        lcopy.start()

        for k in range(K):
            stage_desc(k).wait()
            y_desc(k).start()

        lcopy.wait()

        for k in range(K):
            y_desc(k).wait_recv()
            sl = pl.ds(k * r, r)
            y = res_ref[sl, :] + comm_ref[sl, :]
            ms = jnp.mean(y * y, axis=-1, keepdims=True)
            res_ref[sl, :] = y * lax.rsqrt(ms + 1e-6) * gamma_ref[:]
            x_out_desc(k).start()
            out_desc(k).start()

        for k in range(K):
            x_in_desc(k).wait_recv()
            out_desc(k).wait()
        for k in range(K):
            y_desc(k).wait_send()
            x_out_desc(k).wait_send()

    return pl.pallas_call(
        body,
        out_shape=jax.ShapeDtypeStruct((m_half, d), jnp.float32),
        in_specs=[
            pl.BlockSpec(memory_space=pl.ANY),
            pl.BlockSpec(memory_space=pltpu.VMEM),
        ],
        out_specs=pl.BlockSpec(memory_space=pl.ANY),
        scratch_shapes=[
            pltpu.VMEM((m_q, d), jnp.float32),
            pltpu.VMEM((m_q, d), jnp.float32),
            pltpu.VMEM((m_q, d), jnp.float32),
            pltpu.SemaphoreType.DMA((K,)),
            pltpu.SemaphoreType.DMA((K,)),
            pltpu.SemaphoreType.DMA((K,)),
            pltpu.SemaphoreType.DMA((K,)),
            pltpu.SemaphoreType.DMA,
            pltpu.SemaphoreType.DMA((K,)),
            pltpu.SemaphoreType.DMA((K,)),
        ],
        compiler_params=pltpu.CompilerParams(
            collective_id=0,
            vmem_limit_bytes=56 * 1024 * 1024,
        ),
    )(partial, gamma)


# device time: 116064 ns/iter; 1.0579x vs baseline; 1.0111x over previous
import jax
import jax.numpy as jnp
from jax import lax
from jax.experimental import pallas as pl
from jax.experimental.pallas import tpu as pltpu

K = 32


def kernel(partial, gamma):
    _, m_tot, d = partial.shape
    m_half = m_tot // 2
    m_q = m_half // 2
    r = m_q // K

    def body(partial_ref, gamma_ref, out_ref, comm_ref, stage_ref, res_ref,
             ysend, yrecv, xsend, xrecv, local_sem, stage_sems, out_sems):
        my_x = lax.axis_index("x")
        my_y = lax.axis_index("y")
        y_nbr = (my_x, 1 - my_y)
        x_nbr = (1 - my_x, my_y)

        barrier_sem = pltpu.get_barrier_semaphore()
        for nbr in (y_nbr, x_nbr):
            pl.semaphore_signal(
                barrier_sem, inc=1, device_id=nbr,
                device_id_type=pl.DeviceIdType.MESH,
            )
        pl.semaphore_wait(barrier_sem, 2)

        send_base = (1 - my_y) * m_half + my_x * m_q
        my_base = my_y * m_half + my_x * m_q
        out_q = my_x * m_q

        def y_desc(k):
            return pltpu.make_async_remote_copy(
                src_ref=stage_ref.at[pl.ds(k * r, r), :],
                dst_ref=comm_ref.at[pl.ds(k * r, r), :],
                send_sem=ysend.at[k],
                recv_sem=yrecv.at[k],
                device_id=y_nbr,
                device_id_type=pl.DeviceIdType.MESH,
            )

        def x_out_desc(k):
            return pltpu.make_async_remote_copy(
                src_ref=res_ref.at[pl.ds(k * r, r), :],
                dst_ref=out_ref.at[pl.ds(out_q + k * r, r), :],
                send_sem=xsend.at[k],
                recv_sem=xrecv.at[k],
                device_id=x_nbr,
                device_id_type=pl.DeviceIdType.MESH,
            )

        def x_in_desc(k):
            return pltpu.make_async_remote_copy(
                src_ref=res_ref.at[pl.ds(k * r, r), :],
                dst_ref=out_ref.at[pl.ds((1 - my_x) * m_q + k * r, r), :],
                send_sem=xsend.at[k],
                recv_sem=xrecv.at[k],
                device_id=x_nbr,
                device_id_type=pl.DeviceIdType.MESH,
            )

        def out_desc(k):
            return pltpu.make_async_copy(
                res_ref.at[pl.ds(k * r, r), :],
                out_ref.at[pl.ds(out_q + k * r, r), :],
                out_sems.at[k],
            )

        def stage_desc(k):
            return pltpu.make_async_copy(
                partial_ref.at[0, pl.ds(send_base + k * r, r), :],
                stage_ref.at[pl.ds(k * r, r), :],
                stage_sems.at[k],
            )

        for k in range(K):
            stage_desc(k).start()

        lcopy = pltpu.make_async_copy(
            partial_ref.at[0, pl.ds(my_base, m_q), :],
            res_ref,
            local_sem,
        )
        lcopy.start()

        for k in range(K):
            stage_desc(k).wait()
            y_desc(k).start()

        lcopy.wait()

        for k in range(K):
            y_desc(k).wait_recv()
            sl = pl.ds(k * r, r)
            y = res_ref[sl, :] + comm_ref[sl, :]
            ms = jnp.mean(y * y, axis=-1, keepdims=True)
            res_ref[sl, :] = y * lax.rsqrt(ms + 1e-6) * gamma_ref[:]
            x_out_desc(k).start()
            out_desc(k).start()

        for k in range(K):
            x_in_desc(k).wait_recv()
            out_desc(k).wait()
        for k in range(K):
            y_desc(k).wait_send()
            x_out_desc(k).wait_send()

    return pl.pallas_call(
        body,
        out_shape=jax.ShapeDtypeStruct((m_half, d), jnp.float32),
        in_specs=[
            pl.BlockSpec(memory_space=pl.ANY),
            pl.BlockSpec(memory_space=pltpu.VMEM),
        ],
        out_specs=pl.BlockSpec(memory_space=pl.ANY),
        scratch_shapes=[
            pltpu.VMEM((m_q, d), jnp.float32),
            pltpu.VMEM((m_q, d), jnp.float32),
            pltpu.VMEM((m_q, d), jnp.float32),
            pltpu.SemaphoreType.DMA((K,)),
            pltpu.SemaphoreType.DMA((K,)),
            pltpu.SemaphoreType.DMA((K,)),
            pltpu.SemaphoreType.DMA((K,)),
            pltpu.SemaphoreType.DMA,
            pltpu.SemaphoreType.DMA((K,)),
            pltpu.SemaphoreType.DMA((K,)),
        ],
        compiler_params=pltpu.CompilerParams(
            collective_id=0,
            vmem_limit_bytes=56 * 1024 * 1024,
        ),
    )(partial, gamma)
